# baseline (speedup 1.0000x reference)
import functools

import jax
import jax.numpy as jnp
from jax import lax
from jax.experimental import pallas as pl
from jax.experimental.pallas import tpu as pltpu
from jax.experimental.pallas import tpu_sc as plsc

D_MODEL = 64
NUM_DAYS = 7
NUM_PAIRS = NUM_DAYS * NUM_DAYS
BATCH = 16384
HALF = BATCH // 2
BLK = 8192


@functools.cache
def _build_gather():
    info = plsc.get_sparse_core_info()
    num_cores, num_subcores = info.num_cores, info.num_subcores
    num_workers = num_cores * num_subcores
    p_per_w = HALF // num_workers
    w_per_blk = BLK // p_per_w
    mesh = plsc.VectorSubcoreMesh(core_axis_name="c", subcore_axis_name="s")

    @functools.partial(
        pl.kernel,
        mesh=mesh,
        out_type=jax.ShapeDtypeStruct((HALF, 2 * D_MODEL), jnp.float32),
        scratch_types=[
            pltpu.VMEM((p_per_w,), jnp.int32),
            pltpu.VMEM((p_per_w,), jnp.int32),
            pltpu.VMEM((p_per_w,), jnp.int32),
            pltpu.VMEM((p_per_w, 2 * D_MODEL), jnp.float32),
            pltpu.VMEM_SHARED((NUM_PAIRS, 2 * D_MODEL), jnp.float32),
            pltpu.SemaphoreType.DMA,
        ],
    )
    def gather_kernel(idx_hbm, table2_hbm, out_hbm, ev_v, od_v, pidx_v, rows_v,
                      table2_spm, sem):
        wid = lax.axis_index("s") * num_cores + lax.axis_index("c")
        e_base = (wid // w_per_blk) * (2 * BLK) + (wid % w_per_blk) * p_per_w
        pltpu.sync_copy(idx_hbm.at[pl.ds(e_base, p_per_w)], ev_v)
        pltpu.sync_copy(idx_hbm.at[pl.ds(e_base + BLK, p_per_w)], od_v)

        @pl.when(lax.axis_index("s") == 0)
        def _():
            pltpu.sync_copy(table2_hbm, table2_spm)

        for k in range(p_per_w // 16):
            sl = pl.ds(k * 16, 16)
            pidx_v[sl] = ev_v[sl] * NUM_DAYS + od_v[sl]
        plsc.subcore_barrier()
        pltpu.async_copy(table2_spm.at[pidx_v], rows_v, sem).wait()
        pltpu.sync_copy(rows_v, out_hbm.at[pl.ds(wid * p_per_w, p_per_w)])

    return gather_kernel


def _transpose_body(x_ref, y_ref):
    t = jnp.swapaxes(x_ref[...], 0, 1)
    y_ref[:, :BLK] = t[:D_MODEL]
    y_ref[:, BLK:] = t[D_MODEL:]


@functools.cache
def _build_transpose():
    n_blk = HALF // BLK
    return pl.pallas_call(
        _transpose_body,
        grid=(n_blk,),
        in_specs=[pl.BlockSpec((BLK, 2 * D_MODEL), lambda k: (k, 0))],
        out_specs=pl.BlockSpec((D_MODEL, 2 * BLK), lambda k: (0, k)),
        out_shape=jax.ShapeDtypeStruct((D_MODEL, BATCH), jnp.float32),
    )


def kernel(day_indices, table):
    table2 = jnp.concatenate(
        [jnp.repeat(table, NUM_DAYS, axis=0), jnp.tile(table, (NUM_DAYS, 1))],
        axis=1,
    )
    pairs = _build_gather()(day_indices.astype(jnp.int32), table2)
    y = _build_transpose()(pairs)
    return jnp.transpose(y)


# speedup vs baseline: 1.0667x; 1.0667x over previous
import functools

import jax
import jax.numpy as jnp
from jax import lax
from jax.experimental import pallas as pl
from jax.experimental.pallas import tpu as pltpu
from jax.experimental.pallas import tpu_sc as plsc

D_MODEL = 64
NUM_DAYS = 7
NUM_PAIRS = NUM_DAYS * NUM_DAYS
BATCH = 16384
HALF = BATCH // 2
BLK = 4096


@functools.cache
def _build_gather():
    info = plsc.get_sparse_core_info()
    num_cores, num_subcores = info.num_cores, info.num_subcores
    num_workers = num_cores * num_subcores
    p_per_w = HALF // num_workers
    w_per_blk = BLK // p_per_w
    mesh = plsc.VectorSubcoreMesh(core_axis_name="c", subcore_axis_name="s")

    @functools.partial(
        pl.kernel,
        mesh=mesh,
        out_type=jax.ShapeDtypeStruct((HALF, 2 * D_MODEL), jnp.float32),
        scratch_types=[
            pltpu.VMEM((p_per_w,), jnp.int32),
            pltpu.VMEM((p_per_w,), jnp.int32),
            pltpu.VMEM((p_per_w,), jnp.int32),
            pltpu.VMEM((p_per_w, 2 * D_MODEL), jnp.float32),
            pltpu.VMEM_SHARED((NUM_PAIRS, 2 * D_MODEL), jnp.float32),
            pltpu.SemaphoreType.DMA,
            pltpu.SemaphoreType.DMA,
            pltpu.SemaphoreType.DMA,
            pltpu.SemaphoreType.DMA,
            pltpu.SemaphoreType.DMA,
        ],
    )
    def gather_kernel(idx_hbm, table2_hbm, out_hbm, ev_v, od_v, pidx_v, rows_v,
                      table2_spm, sem_t, sem_e, sem_o, sem_g0, sem_g1):
        wid = lax.axis_index("s") * num_cores + lax.axis_index("c")
        half = p_per_w // 2

        @pl.when(lax.axis_index("s") == 0)
        def _():
            pltpu.async_copy(table2_hbm, table2_spm, sem_t)

        e_base = (wid // w_per_blk) * (2 * BLK) + (wid % w_per_blk) * p_per_w
        e_cp = pltpu.async_copy(idx_hbm.at[pl.ds(e_base, p_per_w)], ev_v, sem_e)
        o_cp = pltpu.async_copy(
            idx_hbm.at[pl.ds(e_base + BLK, p_per_w)], od_v, sem_o)
        e_cp.wait()
        o_cp.wait()
        for k in range(p_per_w // 16):
            sl = pl.ds(k * 16, 16)
            pidx_v[sl] = ev_v[sl] * NUM_DAYS + od_v[sl]

        @pl.when(lax.axis_index("s") == 0)
        def _():
            pltpu.make_async_copy(table2_hbm, table2_spm, sem_t).wait()

        plsc.subcore_barrier()
        g0 = pltpu.async_copy(
            table2_spm.at[pidx_v.at[pl.ds(0, half)]],
            rows_v.at[pl.ds(0, half)], sem_g0)
        g1 = pltpu.async_copy(
            table2_spm.at[pidx_v.at[pl.ds(half, half)]],
            rows_v.at[pl.ds(half, half)], sem_g1)
        base_o = wid * p_per_w
        g0.wait()
        w0 = pltpu.async_copy(
            rows_v.at[pl.ds(0, half)], out_hbm.at[pl.ds(base_o, half)], sem_e)
        g1.wait()
        w1 = pltpu.async_copy(
            rows_v.at[pl.ds(half, half)],
            out_hbm.at[pl.ds(base_o + half, half)], sem_o)
        w0.wait()
        w1.wait()

    return gather_kernel


def _transpose_body(x_ref, y_ref):
    t = jnp.swapaxes(x_ref[...], 0, 1)
    y_ref[:, :BLK] = t[:D_MODEL]
    y_ref[:, BLK:] = t[D_MODEL:]


@functools.cache
def _build_transpose():
    n_blk = HALF // BLK
    return pl.pallas_call(
        _transpose_body,
        grid=(n_blk,),
        in_specs=[pl.BlockSpec((BLK, 2 * D_MODEL), lambda k: (k, 0))],
        out_specs=pl.BlockSpec((D_MODEL, 2 * BLK), lambda k: (0, k)),
        out_shape=jax.ShapeDtypeStruct((D_MODEL, BATCH), jnp.float32),
    )


def kernel(day_indices, table):
    table2 = jnp.concatenate(
        [jnp.repeat(table, NUM_DAYS, axis=0), jnp.tile(table, (NUM_DAYS, 1))],
        axis=1,
    )
    pairs = _build_gather()(day_indices.astype(jnp.int32), table2)
    y = _build_transpose()(pairs)
    return jnp.transpose(y)


# speedup vs baseline: 1.0683x; 1.0015x over previous
import functools

import jax
import jax.numpy as jnp
from jax import lax
from jax.experimental import pallas as pl
from jax.experimental.pallas import tpu as pltpu
from jax.experimental.pallas import tpu_sc as plsc

D_MODEL = 64
NUM_DAYS = 7
NUM_PAIRS = NUM_DAYS * NUM_DAYS
BATCH = 16384
HALF = BATCH // 2
BLK = 4096


@functools.cache
def _build_gather():
    info = plsc.get_sparse_core_info()
    num_cores, num_subcores = info.num_cores, info.num_subcores
    num_workers = num_cores * num_subcores
    p_per_w = HALF // num_workers
    w_per_blk = BLK // p_per_w
    mesh = plsc.VectorSubcoreMesh(core_axis_name="c", subcore_axis_name="s")

    @functools.partial(
        pl.kernel,
        mesh=mesh,
        out_type=jax.ShapeDtypeStruct((HALF, 2 * D_MODEL), jnp.float32),
        scratch_types=[
            pltpu.VMEM((p_per_w,), jnp.int32),
            pltpu.VMEM((p_per_w,), jnp.int32),
            pltpu.VMEM((p_per_w,), jnp.int32),
            pltpu.VMEM((p_per_w, 2 * D_MODEL), jnp.float32),
            pltpu.VMEM_SHARED((NUM_PAIRS, 2 * D_MODEL), jnp.float32),
            pltpu.SemaphoreType.DMA,
            pltpu.SemaphoreType.DMA,
            pltpu.SemaphoreType.DMA,
            pltpu.SemaphoreType.DMA,
            pltpu.SemaphoreType.DMA,
        ],
    )
    def gather_kernel(idx_hbm, table2_hbm, out_hbm, ev_v, od_v, pidx_v, rows_v,
                      table2_spm, sem_t, sem_e, sem_o, sem_g0, sem_g1):
        wid = lax.axis_index("s") * num_cores + lax.axis_index("c")
        half = p_per_w // 2

        @pl.when(lax.axis_index("s") == 0)
        def _():
            pltpu.async_copy(table2_hbm, table2_spm, sem_t)

        e_base = (wid // w_per_blk) * (2 * BLK) + (wid % w_per_blk) * p_per_w
        e_cp = pltpu.async_copy(idx_hbm.at[pl.ds(e_base, p_per_w)], ev_v, sem_e)
        o_cp = pltpu.async_copy(
            idx_hbm.at[pl.ds(e_base + BLK, p_per_w)], od_v, sem_o)
        e_cp.wait()
        o_cp.wait()
        for k in range(p_per_w // 16):
            sl = pl.ds(k * 16, 16)
            pidx_v[sl] = ev_v[sl] * NUM_DAYS + od_v[sl]

        @pl.when(lax.axis_index("s") == 0)
        def _():
            pltpu.make_async_copy(table2_hbm, table2_spm, sem_t).wait()

        plsc.subcore_barrier()
        g0 = pltpu.async_copy(
            table2_spm.at[pidx_v.at[pl.ds(0, half)]],
            rows_v.at[pl.ds(0, half)], sem_g0)
        g1 = pltpu.async_copy(
            table2_spm.at[pidx_v.at[pl.ds(half, half)]],
            rows_v.at[pl.ds(half, half)], sem_g1)
        base_o = wid * p_per_w
        g0.wait()
        w0 = pltpu.async_copy(
            rows_v.at[pl.ds(0, half)], out_hbm.at[pl.ds(base_o, half)], sem_e)
        g1.wait()
        w1 = pltpu.async_copy(
            rows_v.at[pl.ds(half, half)],
            out_hbm.at[pl.ds(base_o + half, half)], sem_o)
        w0.wait()
        w1.wait()

    return gather_kernel


def _transpose_body(x_ref, y_ref):
    h = BLK // 2
    t0 = jnp.swapaxes(x_ref[:h, :], 0, 1)
    t1 = jnp.swapaxes(x_ref[h:, :], 0, 1)
    y_ref[:, 0:h] = t0[:D_MODEL]
    y_ref[:, h:BLK] = t1[:D_MODEL]
    y_ref[:, BLK:BLK + h] = t0[D_MODEL:]
    y_ref[:, BLK + h:] = t1[D_MODEL:]


@functools.cache
def _build_transpose():
    n_blk = HALF // BLK
    return pl.pallas_call(
        _transpose_body,
        grid=(n_blk,),
        in_specs=[pl.BlockSpec((BLK, 2 * D_MODEL), lambda k: (k, 0))],
        out_specs=pl.BlockSpec((D_MODEL, 2 * BLK), lambda k: (0, k)),
        out_shape=jax.ShapeDtypeStruct((D_MODEL, BATCH), jnp.float32),
    )


def kernel(day_indices, table):
    table2 = jnp.concatenate(
        [jnp.repeat(table, NUM_DAYS, axis=0), jnp.tile(table, (NUM_DAYS, 1))],
        axis=1,
    )
    pairs = _build_gather()(day_indices.astype(jnp.int32), table2)
    y = _build_transpose()(pairs)
    return jnp.transpose(y)
